# Initial kernel scaffold; baseline (speedup 1.0000x reference)
import jax
import jax.numpy as jnp
from jax.experimental import pallas as pl


def kernel(seq, W0, W1, W2, W3, W4):
    raise NotImplementedError("write your pallas kernel here")


# speedup vs baseline: 28.6617x; 28.6617x over previous
import functools

import jax
import jax.numpy as jnp
from jax import lax
from jax.experimental import pallas as pl
from jax.experimental.pallas import tpu as pltpu
from jax.experimental.pallas import tpu_sc as plsc

B, L, T, E = 4096, 50, 20, 32
NLAYER = 5
PK = 256
NV = PK // 16
NU = (NLAYER * E) // 16
P = B * L
NW = 32
PW = P // NW
C = 16
IDX = C * T
GW = 64
NG = IDX // GW
NCHUNK = PW // C


def _sc_sums(seq_flat, packed):
    mesh = plsc.VectorSubcoreMesh(core_axis_name="c", subcore_axis_name="s")
    out_ty = [jax.ShapeDtypeStruct((P, E), jnp.float32) for _ in range(NLAYER)]

    @functools.partial(
        pl.kernel,
        mesh=mesh,
        out_type=out_ty,
        scratch_types=[
            pltpu.VMEM((IDX,), jnp.int32),
            pltpu.VMEM((IDX, PK), jnp.float32),
            pltpu.VMEM((NLAYER, C, E), jnp.float32),
            pltpu.SemaphoreType.DMA,
        ],
    )
    def kern(seq_hbm, tab, o0, o1, o2, o3, o4, idx_v, rows_v, s_v, sem):
        wid = lax.axis_index("s") * 2 + lax.axis_index("c")
        outs = [o0, o1, o2, o3, o4]

        def chunk_body(kk, carry):
            base = wid * PW + kk * C
            pltpu.sync_copy(seq_hbm.at[pl.ds(base * T, IDX)], idx_v)
            cps = [
                pltpu.async_copy(
                    tab.at[idx_v.at[pl.ds(j * GW, GW)]],
                    rows_v.at[pl.ds(j * GW, GW)],
                    sem,
                )
                for j in range(NG)
            ]
            for cp in cps:
                cp.wait()

            def pos_body(p, c2):
                r = p * T
                accs = [rows_v[r, pl.ds(16 * k, 16)] for k in range(NU)]
                for t in range(1, T):
                    for k in range(NU):
                        accs[k] = accs[k] + rows_v[r + t, pl.ds(16 * k, 16)]
                for i in range(NLAYER):
                    s_v[i, p, pl.ds(0, 16)] = accs[2 * i]
                    s_v[i, p, pl.ds(16, 16)] = accs[2 * i + 1]
                return c2

            lax.fori_loop(0, C, pos_body, 0)
            for i in range(NLAYER):
                pltpu.sync_copy(s_v.at[i], outs[i].at[pl.ds(base, C)])
            return carry

        lax.fori_loop(0, NCHUNK, chunk_body, 0)

    return kern(seq_flat, packed)


def _finish(seq, s0, s1, s2, s3, s4):
    bblk = 128
    nblk = B // bblk

    def body(seq_ref, r0, r1, r2, r3, r4, out_ref):
        sq = seq_ref[...]
        cnt = jnp.sum((sq != 0).astype(jnp.float32), axis=-1)
        c = cnt[:, :, None]
        nz = c != 0.0
        safe = jnp.where(nz, c, 1.0)
        ms = [jnp.where(nz, r[...] / safe, 0.0) for r in (r0, r1, r2, r3, r4)]
        z1 = jnp.zeros((bblk, 1, E), jnp.float32)
        z2 = jnp.zeros((bblk, 2, E), jnp.float32)
        h = (jnp.concatenate([z2, ms[0][:, :L - 2]], axis=1)
             + jnp.concatenate([z1, ms[1][:, :L - 1]], axis=1)
             + ms[2]
             + jnp.concatenate([ms[3][:, 1:], z1], axis=1)
             + jnp.concatenate([ms[4][:, 2:], z2], axis=1))
        out_ref[...] = jnp.tanh(h)

    seq_spec = pl.BlockSpec((bblk, L, T), lambda b: (b, 0, 0))
    ble_spec = pl.BlockSpec((bblk, L, E), lambda b: (b, 0, 0))
    return pl.pallas_call(
        body,
        grid=(nblk,),
        in_specs=[seq_spec] + [ble_spec] * NLAYER,
        out_specs=ble_spec,
        out_shape=jax.ShapeDtypeStruct((B, L, E), jnp.float32),
    )(seq, s0, s1, s2, s3, s4)


def kernel(seq, W0, W1, W2, W3, W4):
    zrow = jnp.zeros((1, E), jnp.float32)
    tabs = [jnp.concatenate([zrow, W], axis=0) for W in (W0, W1, W2, W3, W4)]
    packed = jnp.concatenate(
        tabs + [jnp.zeros((tabs[0].shape[0], PK - NLAYER * E), jnp.float32)],
        axis=1,
    )
    ss = _sc_sums(seq.reshape(-1), packed)
    ss = [s.reshape(B, L, E) for s in ss]
    return _finish(seq, *ss)


# speedup vs baseline: 47.0633x; 1.6420x over previous
import functools

import jax
import jax.numpy as jnp
from jax import lax
from jax.experimental import pallas as pl
from jax.experimental.pallas import tpu as pltpu
from jax.experimental.pallas import tpu_sc as plsc

B, L, T, E = 4096, 50, 20, 32
NLAYER = 5
PK = 256
NU = (NLAYER * E) // 16
P = B * L
NW = 32
PW = P // NW
C = 8
IDX = C * T
GW = 80
NG = IDX // GW
NCHUNK = PW // C


def _sc_sums(seq_flat, packed):
    mesh = plsc.VectorSubcoreMesh(core_axis_name="c", subcore_axis_name="s")
    out_ty = [jax.ShapeDtypeStruct((P, E), jnp.float32) for _ in range(NLAYER)]

    @functools.partial(
        pl.kernel,
        mesh=mesh,
        out_type=out_ty,
        scratch_types=[
            pltpu.VMEM((IDX,), jnp.int32),
            pltpu.VMEM((IDX,), jnp.int32),
            pltpu.VMEM((IDX, PK), jnp.float32),
            pltpu.VMEM((IDX, PK), jnp.float32),
            pltpu.VMEM((NLAYER, C, E), jnp.float32),
            pltpu.VMEM((NLAYER, C, E), jnp.float32),
            pltpu.SemaphoreType.DMA,
            pltpu.SemaphoreType.DMA,
            pltpu.SemaphoreType.DMA,
            pltpu.SemaphoreType.DMA,
            pltpu.SemaphoreType.DMA,
            pltpu.SemaphoreType.DMA,
        ],
    )
    def kern(seq_hbm, tab, o0, o1, o2, o3, o4,
             idx_v0, idx_v1, rows_v0, rows_v1, s_v0, s_v1,
             isem0, isem1, rsem0, rsem1, osem0, osem1):
        wid = lax.axis_index("s") * 2 + lax.axis_index("c")
        base0 = wid * PW * T
        outs = [o0, o1, o2, o3, o4]
        idxs = [idx_v0, idx_v1]
        rows = [rows_v0, rows_v1]
        svs = [s_v0, s_v1]
        isems = [isem0, isem1]
        rsems = [rsem0, rsem1]
        osems = [osem0, osem1]

        def idx_copy(p, k):
            return pltpu.make_async_copy(
                seq_hbm.at[pl.ds(base0 + k * IDX, IDX)], idxs[p], isems[p])

        def row_copy(p, j):
            return pltpu.make_async_copy(
                tab.at[idxs[p].at[pl.ds(j * GW, GW)]],
                rows[p].at[pl.ds(j * GW, GW)],
                rsems[p])

        def out_copy(p, i, k):
            return pltpu.make_async_copy(
                svs[p].at[i], outs[i].at[pl.ds(wid * PW + k * C, C)], osems[p])

        def fire_rows(p):
            for j in range(NG):
                row_copy(p, j).start()

        def drain_rows(p):
            for j in range(NG):
                row_copy(p, j).wait()

        def reduce_chunk(p):
            rv, sv = rows[p], svs[p]

            def pos_body(q, c2):
                r = q * T
                accs = [rv[r, pl.ds(16 * k, 16)] for k in range(NU)]
                for t in range(1, T):
                    for k in range(NU):
                        accs[k] = accs[k] + rv[r + t, pl.ds(16 * k, 16)]
                for i in range(NLAYER):
                    sv[i, q, pl.ds(0, 16)] = accs[2 * i]
                    sv[i, q, pl.ds(16, 16)] = accs[2 * i + 1]
                return c2

            lax.fori_loop(0, C, pos_body, 0)

        idx_copy(0, 0).start()
        idx_copy(0, 0).wait()
        fire_rows(0)
        idx_copy(1, 1).start()

        def phase(p, k):
            @pl.when(k + 1 < NCHUNK)
            def _():
                idx_copy(1 - p, k + 1).wait()
            drain_rows(p)

            @pl.when(k + 1 < NCHUNK)
            def _():
                fire_rows(1 - p)

            @pl.when(k + 2 < NCHUNK)
            def _():
                idx_copy(p, k + 2).start()

            @pl.when(k >= 2)
            def _():
                for i in range(NLAYER):
                    out_copy(p, i, k - 2).wait()
            reduce_chunk(p)
            for i in range(NLAYER):
                out_copy(p, i, k).start()

        def pair_body(kk, carry):
            phase(0, 2 * kk)
            phase(1, 2 * kk + 1)
            return carry

        lax.fori_loop(0, NCHUNK // 2, pair_body, 0)
        for i in range(NLAYER):
            out_copy(0, i, NCHUNK - 2).wait()
            out_copy(1, i, NCHUNK - 1).wait()

    return kern(seq_flat, packed)


def _finish(seq2d, s0, s1, s2, s3, s4):
    blk = 6400
    nblk = P // blk

    def body(seq_ref, r0, r1, r2, r3, r4, out_ref):
        sq = seq_ref[...]
        cnt = jnp.sum((sq != 0).astype(jnp.float32), axis=-1,
                      keepdims=True)
        nz = cnt != 0.0
        safe = jnp.where(nz, cnt, 1.0)
        ms = [jnp.where(nz, r[...] / safe, 0.0)
              for r in (r0, r1, r2, r3, r4)]
        lmod = lax.broadcasted_iota(jnp.int32, (blk, 1), 0) % L
        z1 = jnp.zeros((1, E), jnp.float32)
        z2 = jnp.zeros((2, E), jnp.float32)
        t0 = jnp.where(lmod >= 2,
                       jnp.concatenate([z2, ms[0][:blk - 2]], axis=0), 0.0)
        t1 = jnp.where(lmod >= 1,
                       jnp.concatenate([z1, ms[1][:blk - 1]], axis=0), 0.0)
        t3 = jnp.where(lmod < L - 1,
                       jnp.concatenate([ms[3][1:], z1], axis=0), 0.0)
        t4 = jnp.where(lmod < L - 2,
                       jnp.concatenate([ms[4][2:], z2], axis=0), 0.0)
        out_ref[...] = jnp.tanh(t0 + t1 + ms[2] + t3 + t4)

    seq_spec = pl.BlockSpec((blk, T), lambda b: (b, 0))
    ble_spec = pl.BlockSpec((blk, E), lambda b: (b, 0))
    return pl.pallas_call(
        body,
        grid=(nblk,),
        in_specs=[seq_spec] + [ble_spec] * NLAYER,
        out_specs=ble_spec,
        out_shape=jax.ShapeDtypeStruct((P, E), jnp.float32),
    )(seq2d, s0, s1, s2, s3, s4)


def kernel(seq, W0, W1, W2, W3, W4):
    zrow = jnp.zeros((1, E), jnp.float32)
    tabs = [jnp.concatenate([zrow, W], axis=0) for W in (W0, W1, W2, W3, W4)]
    packed = jnp.concatenate(
        tabs + [jnp.zeros((tabs[0].shape[0], PK - NLAYER * E), jnp.float32)],
        axis=1,
    )
    seq2d = seq.reshape(P, T)
    ss = _sc_sums(seq2d.reshape(-1), packed)
    return _finish(seq2d, *ss).reshape(B, L, E)


# speedup vs baseline: 50.7811x; 1.0790x over previous
import functools

import jax
import jax.numpy as jnp
from jax import lax
from jax.experimental import pallas as pl
from jax.experimental.pallas import tpu as pltpu
from jax.experimental.pallas import tpu_sc as plsc

B, L, T, E = 4096, 50, 20, 32
NLAYER = 5
TRI = 100000
PK = 256
NU = NLAYER * 2 + 1
P = B * L
NW = 32
PW = P // NW
C = 8
IDX = C * T
GW = 80
NG = IDX // GW
NCHUNK = PW // C


def _pack(w0, w1, w2, w3, w4):
    rblk = 5000
    nblk = TRI // rblk

    def body(a0, a1, a2, a3, a4, out_ref):
        rid = (lax.broadcasted_iota(jnp.int32, (rblk, 1), 0)
               + pl.program_id(0) * rblk)
        valid = rid <= TRI - 2
        for i, a in enumerate((a0, a1, a2, a3, a4)):
            out_ref[:, i * E:(i + 1) * E] = jnp.where(valid, a[...], 0.0)
        out_ref[:, NLAYER * E:NLAYER * E + 16] = jnp.where(
            valid, jnp.ones((rblk, 16), jnp.float32), 0.0)
        out_ref[:, NLAYER * E + 16:] = jnp.zeros(
            (rblk, PK - NLAYER * E - 16), jnp.float32)

    w_spec = pl.BlockSpec((rblk, E), lambda b: (b, 0))
    return pl.pallas_call(
        body,
        grid=(nblk,),
        in_specs=[w_spec] * NLAYER,
        out_specs=pl.BlockSpec((rblk, PK), lambda b: (b, 0)),
        out_shape=jax.ShapeDtypeStruct((TRI, PK), jnp.float32),
    )(w0, w1, w2, w3, w4)


def _sc_sums(seq_flat, packed):
    mesh = plsc.VectorSubcoreMesh(core_axis_name="c", subcore_axis_name="s")
    out_ty = ([jax.ShapeDtypeStruct((P, E), jnp.float32)
               for _ in range(NLAYER)]
              + [jax.ShapeDtypeStruct((P, 16), jnp.float32)])

    @functools.partial(
        pl.kernel,
        mesh=mesh,
        out_type=out_ty,
        scratch_types=[
            pltpu.VMEM((IDX,), jnp.int32),
            pltpu.VMEM((IDX,), jnp.int32),
            pltpu.VMEM((IDX, PK), jnp.float32),
            pltpu.VMEM((IDX, PK), jnp.float32),
            pltpu.VMEM((NLAYER, C, E), jnp.float32),
            pltpu.VMEM((NLAYER, C, E), jnp.float32),
            pltpu.VMEM((C, 16), jnp.float32),
            pltpu.VMEM((C, 16), jnp.float32),
            pltpu.SemaphoreType.DMA,
            pltpu.SemaphoreType.DMA,
            pltpu.SemaphoreType.DMA,
            pltpu.SemaphoreType.DMA,
            pltpu.SemaphoreType.DMA,
            pltpu.SemaphoreType.DMA,
        ],
    )
    def kern(seq_hbm, tab, o0, o1, o2, o3, o4, ocnt,
             idx_v0, idx_v1, rows_v0, rows_v1, s_v0, s_v1, c_v0, c_v1,
             isem0, isem1, rsem0, rsem1, osem0, osem1):
        wid = lax.axis_index("s") * 2 + lax.axis_index("c")
        base0 = wid * PW * T
        outs = [o0, o1, o2, o3, o4]
        idxs = [idx_v0, idx_v1]
        rows = [rows_v0, rows_v1]
        svs = [s_v0, s_v1]
        cvs = [c_v0, c_v1]
        isems = [isem0, isem1]
        rsems = [rsem0, rsem1]
        osems = [osem0, osem1]

        def idx_copy(p, k):
            return pltpu.make_async_copy(
                seq_hbm.at[pl.ds(base0 + k * IDX, IDX)], idxs[p], isems[p])

        def adjust_idx(p):
            iv = idxs[p]
            for v in range(IDX // 16):
                x = iv[pl.ds(16 * v, 16)]
                iv[pl.ds(16 * v, 16)] = jnp.where(x == 0, TRI - 1, x - 1)

        def row_copy(p, j):
            return pltpu.make_async_copy(
                tab.at[idxs[p].at[pl.ds(j * GW, GW)]],
                rows[p].at[pl.ds(j * GW, GW)],
                rsems[p])

        def out_copy(p, i, k):
            if i == NLAYER:
                return pltpu.make_async_copy(
                    cvs[p], ocnt.at[pl.ds(wid * PW + k * C, C)], osems[p])
            return pltpu.make_async_copy(
                svs[p].at[i], outs[i].at[pl.ds(wid * PW + k * C, C)], osems[p])

        def fire_rows(p):
            for j in range(NG):
                row_copy(p, j).start()

        def drain_rows(p):
            for j in range(NG):
                row_copy(p, j).wait()

        def reduce_chunk(p):
            rv, sv, cv = rows[p], svs[p], cvs[p]

            def pos_body(q, c2):
                r = q * T
                accs = [rv[r, pl.ds(16 * k, 16)] for k in range(NU)]
                for t in range(1, T):
                    for k in range(NU):
                        accs[k] = accs[k] + rv[r + t, pl.ds(16 * k, 16)]
                for i in range(NLAYER):
                    sv[i, q, pl.ds(0, 16)] = accs[2 * i]
                    sv[i, q, pl.ds(16, 16)] = accs[2 * i + 1]
                cv[q, pl.ds(0, 16)] = accs[NU - 1]
                return c2

            lax.fori_loop(0, C, pos_body, 0)

        idx_copy(0, 0).start()
        idx_copy(0, 0).wait()
        adjust_idx(0)
        fire_rows(0)
        idx_copy(1, 1).start()

        def phase(p, k):
            @pl.when(k + 1 < NCHUNK)
            def _():
                idx_copy(1 - p, k + 1).wait()
                adjust_idx(1 - p)
            drain_rows(p)

            @pl.when(k + 1 < NCHUNK)
            def _():
                fire_rows(1 - p)

            @pl.when(k + 2 < NCHUNK)
            def _():
                idx_copy(p, k + 2).start()

            @pl.when(k >= 2)
            def _():
                for i in range(NLAYER + 1):
                    out_copy(p, i, k - 2).wait()
            reduce_chunk(p)
            for i in range(NLAYER + 1):
                out_copy(p, i, k).start()

        def pair_body(kk, carry):
            phase(0, 2 * kk)
            phase(1, 2 * kk + 1)
            return carry

        lax.fori_loop(0, NCHUNK // 2, pair_body, 0)
        for i in range(NLAYER + 1):
            out_copy(0, i, NCHUNK - 2).wait()
            out_copy(1, i, NCHUNK - 1).wait()

    return kern(seq_flat, packed)


def _finish(s0, s1, s2, s3, s4, rcnt):
    blk = 6400
    nblk = P // blk

    def body(r0, r1, r2, r3, r4, rc, out_ref):
        c16 = rc[...]
        c32 = jnp.concatenate([c16, c16], axis=1)
        nz = c32 != 0.0
        safe = jnp.where(nz, c32, 1.0)
        ms = [jnp.where(nz, r[...] / safe, 0.0)
              for r in (r0, r1, r2, r3, r4)]
        lmod = lax.broadcasted_iota(jnp.int32, (blk, 1), 0) % L
        z1 = jnp.zeros((1, E), jnp.float32)
        z2 = jnp.zeros((2, E), jnp.float32)
        t0 = jnp.where(lmod >= 2,
                       jnp.concatenate([z2, ms[0][:blk - 2]], axis=0), 0.0)
        t1 = jnp.where(lmod >= 1,
                       jnp.concatenate([z1, ms[1][:blk - 1]], axis=0), 0.0)
        t3 = jnp.where(lmod < L - 1,
                       jnp.concatenate([ms[3][1:], z1], axis=0), 0.0)
        t4 = jnp.where(lmod < L - 2,
                       jnp.concatenate([ms[4][2:], z2], axis=0), 0.0)
        out_ref[...] = jnp.tanh(t0 + t1 + ms[2] + t3 + t4)

    ble_spec = pl.BlockSpec((blk, E), lambda b: (b, 0))
    cnt_spec = pl.BlockSpec((blk, 16), lambda b: (b, 0))
    return pl.pallas_call(
        body,
        grid=(nblk,),
        in_specs=[ble_spec] * NLAYER + [cnt_spec],
        out_specs=ble_spec,
        out_shape=jax.ShapeDtypeStruct((P, E), jnp.float32),
    )(s0, s1, s2, s3, s4, rcnt)


def kernel(seq, W0, W1, W2, W3, W4):
    packed = _pack(W0, W1, W2, W3, W4)
    outs = _sc_sums(seq.reshape(-1), packed)
    return _finish(*outs).reshape(B, L, E)


# speedup vs baseline: 65.7722x; 1.2952x over previous
import functools

import jax
import jax.numpy as jnp
from jax import lax
from jax.experimental import pallas as pl
from jax.experimental.pallas import tpu as pltpu
from jax.experimental.pallas import tpu_sc as plsc

B, L, T, E = 4096, 50, 20, 32
NLAYER = 5
TRI = 100000
PK = 256
NU = NLAYER * 2 + 1
P = B * L
NW = 32
PW = P // NW
C = 8
IDX = C * T
GW = 80
NG = IDX // GW
NCHUNK = PW // C


def _pack(w0, w1, w2, w3, w4):
    rblk = 5000
    nblk = TRI // rblk

    def body(a0, a1, a2, a3, a4, out_ref):
        rid = (lax.broadcasted_iota(jnp.int32, (rblk, 1), 0)
               + pl.program_id(0) * rblk)
        valid = rid <= TRI - 2
        for i, a in enumerate((a0, a1, a2, a3, a4)):
            wb = jnp.where(valid, a[...], 0.0).astype(jnp.bfloat16)
            lo = lax.bitcast_convert_type(
                wb[:, 0:16], jnp.uint16).astype(jnp.uint32)
            hi = lax.bitcast_convert_type(
                wb[:, 16:32], jnp.uint16).astype(jnp.uint32)
            word = (lo | (hi << 16)).astype(jnp.int32)
            out_ref[:, i * 16:(i + 1) * 16] = word
        ones_word = jnp.int32(0x3F803F80)
        out_ref[:, 80:96] = jnp.where(
            valid, jnp.full((rblk, 16), ones_word, jnp.int32), 0)
        out_ref[:, 96:128] = jnp.zeros((rblk, 32), jnp.int32)

    w_spec = pl.BlockSpec((rblk, E), lambda b: (b, 0))
    return pl.pallas_call(
        body,
        grid=(nblk,),
        in_specs=[w_spec] * NLAYER,
        out_specs=pl.BlockSpec((rblk, 128), lambda b: (b, 0)),
        out_shape=jax.ShapeDtypeStruct((TRI, 128), jnp.int32),
    )(w0, w1, w2, w3, w4)


def _sc_sums(seq_flat, packed):
    mesh = plsc.VectorSubcoreMesh(core_axis_name="c", subcore_axis_name="s")
    out_ty = ([jax.ShapeDtypeStruct((P, E), jnp.float32)
               for _ in range(NLAYER)]
              + [jax.ShapeDtypeStruct((P, 16), jnp.float32)])

    @functools.partial(
        pl.kernel,
        mesh=mesh,
        out_type=out_ty,
        scratch_types=[
            pltpu.VMEM((IDX,), jnp.int32),
            pltpu.VMEM((IDX,), jnp.int32),
            pltpu.VMEM((IDX, 128), jnp.int32),
            pltpu.VMEM((IDX, 128), jnp.int32),
            pltpu.VMEM((NLAYER, C, E), jnp.float32),
            pltpu.VMEM((NLAYER, C, E), jnp.float32),
            pltpu.VMEM((C, 16), jnp.float32),
            pltpu.VMEM((C, 16), jnp.float32),
            pltpu.SemaphoreType.DMA,
            pltpu.SemaphoreType.DMA,
            pltpu.SemaphoreType.DMA,
            pltpu.SemaphoreType.DMA,
            pltpu.SemaphoreType.DMA,
            pltpu.SemaphoreType.DMA,
        ],
    )
    def kern(seq_hbm, tab, o0, o1, o2, o3, o4, ocnt,
             idx_v0, idx_v1, rows_v0, rows_v1, s_v0, s_v1, c_v0, c_v1,
             isem0, isem1, rsem0, rsem1, osem0, osem1):
        wid = lax.axis_index("s") * 2 + lax.axis_index("c")
        base0 = wid * PW * T
        outs = [o0, o1, o2, o3, o4, ocnt]
        idxs = [idx_v0, idx_v1]
        rows = [rows_v0, rows_v1]
        svs = [s_v0, s_v1]
        cvs = [c_v0, c_v1]
        isems = [isem0, isem1]
        rsems = [rsem0, rsem1]
        osems = [osem0, osem1]

        def idx_copy(p, k):
            return pltpu.make_async_copy(
                seq_hbm.at[pl.ds(base0 + k * IDX, IDX)], idxs[p], isems[p])

        def adjust_idx(p):
            iv = idxs[p]
            for v in range(IDX // 16):
                x = iv[pl.ds(16 * v, 16)]
                iv[pl.ds(16 * v, 16)] = jnp.where(x == 0, TRI - 1, x - 1)

        def row_copy(p, j):
            return pltpu.make_async_copy(
                tab.at[idxs[p].at[pl.ds(j * GW, GW)]],
                rows[p].at[pl.ds(j * GW, GW)],
                rsems[p])

        def out_copy(p, i, k):
            src = cvs[p] if i == NLAYER else svs[p].at[i]
            return pltpu.make_async_copy(
                src, outs[i].at[pl.ds(wid * PW + k * C, C)], osems[p])

        def fire_rows(p):
            for j in range(NG):
                row_copy(p, j).start()

        def drain_rows(p):
            for j in range(NG):
                row_copy(p, j).wait()

        himask = jnp.int32(-65536)

        def reduce_chunk(p):
            rv, sv, cv = rows[p], svs[p], cvs[p]

            def pos_body(q, c2):
                r = q * T
                for si in range(NLAYER + 1):
                    acc_e = acc_o = None
                    for t in range(T):
                        x = rv[r + t, pl.ds(si * 16, 16)]
                        ye = lax.bitcast_convert_type(x << 16, jnp.float32)
                        acc_e = ye if acc_e is None else acc_e + ye
                        if si < NLAYER:
                            yo = lax.bitcast_convert_type(x & himask, jnp.float32)
                            acc_o = yo if acc_o is None else acc_o + yo
                    if si < NLAYER:
                        sv[si, q, pl.ds(0, 16)] = acc_e
                        sv[si, q, pl.ds(16, 16)] = acc_o
                    else:
                        cv[q, pl.ds(0, 16)] = acc_e
                return c2

            lax.fori_loop(0, C, pos_body, 0)

        idx_copy(0, 0).start()
        idx_copy(0, 0).wait()
        adjust_idx(0)
        fire_rows(0)
        idx_copy(1, 1).start()

        def phase(p, k):
            @pl.when(k + 1 < NCHUNK)
            def _():
                idx_copy(1 - p, k + 1).wait()
                adjust_idx(1 - p)
            drain_rows(p)

            @pl.when(k + 1 < NCHUNK)
            def _():
                fire_rows(1 - p)

            @pl.when(k + 2 < NCHUNK)
            def _():
                idx_copy(p, k + 2).start()

            @pl.when(k >= 2)
            def _():
                for i in range(NLAYER + 1):
                    out_copy(p, i, k - 2).wait()
            reduce_chunk(p)
            for i in range(NLAYER + 1):
                out_copy(p, i, k).start()

        def pair_body(kk, carry):
            phase(0, 2 * kk)
            phase(1, 2 * kk + 1)
            return carry

        lax.fori_loop(0, NCHUNK // 2, pair_body, 0)
        for i in range(NLAYER + 1):
            out_copy(0, i, NCHUNK - 2).wait()
            out_copy(1, i, NCHUNK - 1).wait()

    return kern(seq_flat, packed)


def _finish(s0, s1, s2, s3, s4, rcnt):
    blk = 6400
    nblk = P // blk

    def body(r0, r1, r2, r3, r4, rc, out_ref):
        c16 = rc[...]
        c32 = jnp.concatenate([c16, c16], axis=1)
        nz = c32 != 0.0
        safe = jnp.where(nz, c32, 1.0)
        ms = [jnp.where(nz, r[...] / safe, 0.0)
              for r in (r0, r1, r2, r3, r4)]
        lmod = lax.broadcasted_iota(jnp.int32, (blk, 1), 0) % L
        z1 = jnp.zeros((1, E), jnp.float32)
        z2 = jnp.zeros((2, E), jnp.float32)
        t0 = jnp.where(lmod >= 2,
                       jnp.concatenate([z2, ms[0][:blk - 2]], axis=0), 0.0)
        t1 = jnp.where(lmod >= 1,
                       jnp.concatenate([z1, ms[1][:blk - 1]], axis=0), 0.0)
        t3 = jnp.where(lmod < L - 1,
                       jnp.concatenate([ms[3][1:], z1], axis=0), 0.0)
        t4 = jnp.where(lmod < L - 2,
                       jnp.concatenate([ms[4][2:], z2], axis=0), 0.0)
        out_ref[...] = jnp.tanh(t0 + t1 + ms[2] + t3 + t4)

    ble_spec = pl.BlockSpec((blk, E), lambda b: (b, 0))
    cnt_spec = pl.BlockSpec((blk, 16), lambda b: (b, 0))
    return pl.pallas_call(
        body,
        grid=(nblk,),
        in_specs=[ble_spec] * NLAYER + [cnt_spec],
        out_specs=ble_spec,
        out_shape=jax.ShapeDtypeStruct((P, E), jnp.float32),
    )(s0, s1, s2, s3, s4, rcnt)


def kernel(seq, W0, W1, W2, W3, W4):
    packed = _pack(W0, W1, W2, W3, W4)
    outs = _sc_sums(seq.reshape(-1), packed)
    return _finish(*outs).reshape(B, L, E)


# speedup vs baseline: 80.7326x; 1.2275x over previous
import functools

import jax
import jax.numpy as jnp
from jax import lax
from jax.experimental import pallas as pl
from jax.experimental.pallas import tpu as pltpu
from jax.experimental.pallas import tpu_sc as plsc

B, L, T, E = 4096, 50, 20, 32
NLAYER = 5
TRI = 100000
PK = 256
NU = NLAYER * 2 + 1
P = B * L
NW = 32
PW = P // NW
C = 16
IDX = C * T
GW = 80
NG = IDX // GW
NCHUNK = PW // C


def _pack(w0, w1, w2, w3, w4):
    rblk = 5000
    nblk = TRI // rblk

    def body(a0, a1, a2, a3, a4, out_ref):
        rid = (lax.broadcasted_iota(jnp.int32, (rblk, 1), 0)
               + pl.program_id(0) * rblk)
        valid = rid <= TRI - 2
        for i, a in enumerate((a0, a1, a2, a3, a4)):
            wb = jnp.where(valid, a[...], 0.0).astype(jnp.bfloat16)
            lo = lax.bitcast_convert_type(
                wb[:, 0:16], jnp.uint16).astype(jnp.uint32)
            hi = lax.bitcast_convert_type(
                wb[:, 16:32], jnp.uint16).astype(jnp.uint32)
            word = (lo | (hi << 16)).astype(jnp.int32)
            out_ref[:, i * 16:(i + 1) * 16] = word
        ones_word = jnp.int32(0x3F803F80)
        out_ref[:, 80:96] = jnp.where(
            valid, jnp.full((rblk, 16), ones_word, jnp.int32), 0)
        out_ref[:, 96:128] = jnp.zeros((rblk, 32), jnp.int32)

    w_spec = pl.BlockSpec((rblk, E), lambda b: (b, 0))
    return pl.pallas_call(
        body,
        grid=(nblk,),
        in_specs=[w_spec] * NLAYER,
        out_specs=pl.BlockSpec((rblk, 128), lambda b: (b, 0)),
        out_shape=jax.ShapeDtypeStruct((TRI, 128), jnp.int32),
    )(w0, w1, w2, w3, w4)


def _sc_sums(seq_flat, packed):
    mesh = plsc.VectorSubcoreMesh(core_axis_name="c", subcore_axis_name="s")
    out_ty = ([jax.ShapeDtypeStruct((P, E), jnp.float32)
               for _ in range(NLAYER)]
              + [jax.ShapeDtypeStruct((P, 16), jnp.float32)])

    @functools.partial(
        pl.kernel,
        mesh=mesh,
        out_type=out_ty,
        scratch_types=[
            pltpu.VMEM((IDX,), jnp.int32),
            pltpu.VMEM((IDX,), jnp.int32),
            pltpu.VMEM((IDX, 128), jnp.int32),
            pltpu.VMEM((IDX, 128), jnp.int32),
            pltpu.VMEM((NLAYER, C, E), jnp.float32),
            pltpu.VMEM((NLAYER, C, E), jnp.float32),
            pltpu.VMEM((C, 16), jnp.float32),
            pltpu.VMEM((C, 16), jnp.float32),
            pltpu.SemaphoreType.DMA,
            pltpu.SemaphoreType.DMA,
            pltpu.SemaphoreType.DMA,
            pltpu.SemaphoreType.DMA,
            pltpu.SemaphoreType.DMA,
            pltpu.SemaphoreType.DMA,
        ],
    )
    def kern(seq_hbm, tab, o0, o1, o2, o3, o4, ocnt,
             idx_v0, idx_v1, rows_v0, rows_v1, s_v0, s_v1, c_v0, c_v1,
             isem0, isem1, rsem0, rsem1, osem0, osem1):
        wid = lax.axis_index("s") * 2 + lax.axis_index("c")
        base0 = wid * PW * T
        outs = [o0, o1, o2, o3, o4, ocnt]
        idxs = [idx_v0, idx_v1]
        rows = [rows_v0, rows_v1]
        svs = [s_v0, s_v1]
        cvs = [c_v0, c_v1]
        isems = [isem0, isem1]
        rsems = [rsem0, rsem1]
        osems = [osem0, osem1]

        def idx_copy(p, k):
            return pltpu.make_async_copy(
                seq_hbm.at[pl.ds(base0 + k * IDX, IDX)], idxs[p], isems[p])

        def adjust_idx(p):
            iv = idxs[p]
            for v in range(IDX // 16):
                x = iv[pl.ds(16 * v, 16)]
                iv[pl.ds(16 * v, 16)] = jnp.where(x == 0, TRI - 1, x - 1)

        def row_copy(p, j):
            return pltpu.make_async_copy(
                tab.at[idxs[p].at[pl.ds(j * GW, GW)]],
                rows[p].at[pl.ds(j * GW, GW)],
                rsems[p])

        def out_copy(p, i, k):
            src = cvs[p] if i == NLAYER else svs[p].at[i]
            return pltpu.make_async_copy(
                src, outs[i].at[pl.ds(wid * PW + k * C, C)], osems[p])

        def fire_rows(p):
            for j in range(NG):
                row_copy(p, j).start()

        def drain_rows(p):
            for j in range(NG):
                row_copy(p, j).wait()

        himask = jnp.int32(-65536)

        def reduce_chunk(p):
            rv, sv, cv = rows[p], svs[p], cvs[p]

            def pos_body(q, c2):
                r = q * T
                for si in range(NLAYER + 1):
                    acc_e = acc_o = None
                    for t in range(T):
                        x = rv[r + t, pl.ds(si * 16, 16)]
                        ye = lax.bitcast_convert_type(x << 16, jnp.float32)
                        acc_e = ye if acc_e is None else acc_e + ye
                        if si < NLAYER:
                            yo = lax.bitcast_convert_type(x & himask, jnp.float32)
                            acc_o = yo if acc_o is None else acc_o + yo
                    if si < NLAYER:
                        sv[si, q, pl.ds(0, 16)] = acc_e
                        sv[si, q, pl.ds(16, 16)] = acc_o
                    else:
                        cv[q, pl.ds(0, 16)] = acc_e
                return c2

            lax.fori_loop(0, C, pos_body, 0)

        idx_copy(0, 0).start()
        idx_copy(0, 0).wait()
        adjust_idx(0)
        fire_rows(0)
        idx_copy(1, 1).start()

        def phase(p, k):
            @pl.when(k + 1 < NCHUNK)
            def _():
                idx_copy(1 - p, k + 1).wait()
                adjust_idx(1 - p)
            drain_rows(p)

            @pl.when(k + 1 < NCHUNK)
            def _():
                fire_rows(1 - p)

            @pl.when(k + 2 < NCHUNK)
            def _():
                idx_copy(p, k + 2).start()

            @pl.when(k >= 2)
            def _():
                for i in range(NLAYER + 1):
                    out_copy(p, i, k - 2).wait()
            reduce_chunk(p)
            for i in range(NLAYER + 1):
                out_copy(p, i, k).start()

        def pair_body(kk, carry):
            phase(0, 2 * kk)
            phase(1, 2 * kk + 1)
            return carry

        lax.fori_loop(0, NCHUNK // 2, pair_body, 0)
        for i in range(NLAYER + 1):
            out_copy(0, i, NCHUNK - 2).wait()
            out_copy(1, i, NCHUNK - 1).wait()

    return kern(seq_flat, packed)


def _finish(s0, s1, s2, s3, s4, rcnt):
    blk = 6400
    nblk = P // blk

    def body(r0, r1, r2, r3, r4, rc, out_ref):
        c16 = rc[...]
        c32 = jnp.concatenate([c16, c16], axis=1)
        nz = c32 != 0.0
        safe = jnp.where(nz, c32, 1.0)
        ms = [jnp.where(nz, r[...] / safe, 0.0)
              for r in (r0, r1, r2, r3, r4)]
        lmod = lax.broadcasted_iota(jnp.int32, (blk, 1), 0) % L
        z1 = jnp.zeros((1, E), jnp.float32)
        z2 = jnp.zeros((2, E), jnp.float32)
        t0 = jnp.where(lmod >= 2,
                       jnp.concatenate([z2, ms[0][:blk - 2]], axis=0), 0.0)
        t1 = jnp.where(lmod >= 1,
                       jnp.concatenate([z1, ms[1][:blk - 1]], axis=0), 0.0)
        t3 = jnp.where(lmod < L - 1,
                       jnp.concatenate([ms[3][1:], z1], axis=0), 0.0)
        t4 = jnp.where(lmod < L - 2,
                       jnp.concatenate([ms[4][2:], z2], axis=0), 0.0)
        h = jnp.tanh(t0 + t1 + ms[2] + t3 + t4)
        out_ref[...] = h.reshape(blk // L, L, E)

    ble_spec = pl.BlockSpec((blk, E), lambda b: (b, 0))
    cnt_spec = pl.BlockSpec((blk, 16), lambda b: (b, 0))
    return pl.pallas_call(
        body,
        grid=(nblk,),
        in_specs=[ble_spec] * NLAYER + [cnt_spec],
        out_specs=pl.BlockSpec((blk // L, L, E), lambda b: (b, 0, 0)),
        out_shape=jax.ShapeDtypeStruct((B, L, E), jnp.float32),
    )(s0, s1, s2, s3, s4, rcnt)


def kernel(seq, W0, W1, W2, W3, W4):
    packed = _pack(W0, W1, W2, W3, W4)
    outs = _sc_sums(seq.reshape(-1), packed)
    return _finish(*outs)
